# Initial kernel scaffold; baseline (speedup 1.0000x reference)
import jax
import jax.numpy as jnp
from jax.experimental import pallas as pl


def kernel(x, edge_index, W1, a_src1, a_dst1, b1, W2, a_src2, a_dst2, b2, W3, a_src3, a_dst3, b3):
    raise NotImplementedError("write your pallas kernel here")


# speedup vs baseline: 1.0000x; 1.0000x over previous
import jax
import jax.numpy as jnp
from jax.experimental import pallas as pl

N = 10000
HEADS = 8
HID = 16
D_OUT = 64


def _gat_layer(x, edge_index, W, a_src, a_dst, b, heads, out_ch):
    n = x.shape[0]
    loop = jnp.arange(n, dtype=edge_index.dtype)
    src = jnp.concatenate([edge_index[0], loop])
    dst = jnp.concatenate([edge_index[1], loop])
    h = (x @ W).reshape(n, heads, out_ch)
    alpha_src = jnp.sum(h * a_src[None, :, :], axis=-1)
    alpha_dst = jnp.sum(h * a_dst[None, :, :], axis=-1)
    e = alpha_src[src] + alpha_dst[dst]
    e = jnp.where(e > 0, e, 0.2 * e)
    m = jax.ops.segment_max(e, dst, num_segments=n)
    ex = jnp.exp(e - m[dst])
    s = jax.ops.segment_sum(ex, dst, num_segments=n)
    alpha = ex / (s[dst] + 1e-16)
    msg = h[src] * alpha[:, :, None]
    out = jax.ops.segment_sum(msg, dst, num_segments=n)
    return out.reshape(n, heads * out_ch) + b


def _log_softmax_body(x_ref, o_ref):
    x = x_ref[...]
    m = jnp.max(x, axis=-1, keepdims=True)
    ex = jnp.exp(x - m)
    o_ref[...] = (x - m) - jnp.log(jnp.sum(ex, axis=-1, keepdims=True))


def _log_softmax(x):
    return pl.pallas_call(
        _log_softmax_body,
        out_shape=jax.ShapeDtypeStruct(x.shape, x.dtype),
    )(x)


def kernel(x, edge_index, W1, a_src1, a_dst1, b1, W2, a_src2, a_dst2, b2,
           W3, a_src3, a_dst3, b3):
    h = jax.nn.relu(_gat_layer(x, edge_index, W1, a_src1, a_dst1, b1, HEADS, HID))
    h = jax.nn.relu(_gat_layer(h, edge_index, W2, a_src2, a_dst2, b2, HEADS, HID))
    h = _gat_layer(h, edge_index, W3, a_src3, a_dst3, b3, 1, D_OUT)
    return _log_softmax(h)


# speedup vs baseline: 15.5671x; 15.5665x over previous
import functools

import jax
import jax.numpy as jnp
from jax import lax
from jax.experimental import pallas as pl
from jax.experimental.pallas import tpu as pltpu
from jax.experimental.pallas import tpu_sc as plsc

N = 10000
HEADS = 8
HID = 16
D_OUT = 64
NW = 32
NPT = 320
NPAD = NW * NPT
CE = 128
LRELU = 0.2
EPS = 1e-16


def _dense_body(kg, x_ref, w_ref, ps_ref, pd_ref, hc_ref, ad_ref, m_ref):
    n = x_ref.shape[0]
    h = jnp.dot(x_ref[...], w_ref[...], preferred_element_type=jnp.float32)
    a_s = jnp.dot(h, ps_ref[...], preferred_element_type=jnp.float32)
    a_d = jnp.dot(h, pd_ref[...], preferred_element_type=jnp.float32)
    k = h.shape[1]
    pad = kg - k - 16
    hc_ref[...] = jnp.concatenate(
        [h, a_s, jnp.zeros((n, pad), jnp.float32)], axis=1)
    ad_ref[...] = a_d
    t = jnp.max(a_s, axis=0, keepdims=True) + jnp.max(a_d, axis=0, keepdims=True)
    m_ref[...] = jnp.maximum(t, LRELU * t)


def _dense(x, w, ps, pd, kg):
    n = x.shape[0]
    return pl.pallas_call(
        functools.partial(_dense_body, kg),
        out_shape=(
            jax.ShapeDtypeStruct((n, kg), jnp.float32),
            jax.ShapeDtypeStruct((n, 16), jnp.float32),
            jax.ShapeDtypeStruct((1, 16), jnp.float32),
        ),
    )(x, w, ps, pd)


def _log_softmax_body(x_ref, o_ref):
    x = x_ref[...]
    m = jnp.max(x, axis=-1, keepdims=True)
    ex = jnp.exp(x - m)
    o_ref[...] = (x - m) - jnp.log(jnp.sum(ex, axis=-1, keepdims=True))


def _log_softmax(x):
    return pl.pallas_call(
        _log_softmax_body,
        out_shape=jax.ShapeDtypeStruct(x.shape, x.dtype),
    )(x)


def _edge_kernel(K, KG, CPH, relu):
    NV = K // 16
    mesh = plsc.VectorSubcoreMesh(core_axis_name="c", subcore_axis_name="s",
                                  num_cores=2, num_subcores=16)

    @functools.partial(
        pl.kernel,
        out_type=jax.ShapeDtypeStruct((NPAD, K), jnp.float32),
        mesh=mesh,
        scratch_types=[
            pltpu.VMEM((16,), jnp.int32),
            pltpu.VMEM((16,), jnp.float32),
            pltpu.VMEM((K,), jnp.float32),
            pltpu.VMEM((CE,), jnp.int32),
            pltpu.VMEM((CE,), jnp.int32),
            pltpu.VMEM((CE, KG), jnp.float32),
            pltpu.VMEM((NPT, 16), jnp.float32),
            pltpu.VMEM((K,), jnp.float32),
            pltpu.VMEM((16,), jnp.float32),
            pltpu.VMEM((NPT, K), jnp.float32),
            pltpu.SemaphoreType.DMA,
        ],
    )
    def k(srcp, dstp, tecb, adst, m16, bias, hcat, out,
          bounds_v, m_v, bias_v, src_v, dst_v, h_v, ad_v,
          acc_v, s_v, stg_v, sem_h):
        t = lax.axis_index("s") * 2 + lax.axis_index("c")
        pltpu.sync_copy(tecb.at[pl.ds(pl.multiple_of(t * 16, 8), 16)], bounds_v)
        pltpu.sync_copy(m16, m_v)
        pltpu.sync_copy(bias, bias_v)
        bv = bounds_v[...]
        elo = bv[0]
        ehi = bv[1]
        nlo = bv[2]
        pltpu.sync_copy(adst.at[pl.ds(pl.multiple_of(nlo, 8), NPT)], ad_v)
        mm = m_v[...]
        zero16 = jnp.zeros((16,), jnp.float32)
        s_v[...] = zero16
        for j in range(NV):
            acc_v[pl.ds(j * 16, 16)] = zero16

        def finalize(cur):
            recip = 1.0 / (s_v[...] + EPS)
            row = cur - nlo
            for j in range(NV):
                rj = recip[(j * 16) // CPH]
                ov = acc_v[pl.ds(j * 16, 16)] * rj + bias_v[pl.ds(j * 16, 16)]
                if relu:
                    ov = jnp.maximum(ov, 0.0)
                stg_v[row, pl.ds(j * 16, 16)] = ov
                acc_v[pl.ds(j * 16, 16)] = zero16
            s_v[...] = zero16

        nchunks = (ehi - elo + CE - 1) // CE

        def chunk_body(c, cur):
            base = pl.multiple_of(elo + c * CE, 8)
            pltpu.sync_copy(srcp.at[pl.ds(base, CE)], src_v)
            pltpu.sync_copy(dstp.at[pl.ds(base, CE)], dst_v)
            pltpu.async_copy(hcat.at[src_v], h_v, sem_h).wait()

            nvalid = jnp.minimum(CE, ehi - base)
            ngroups = (nvalid + 15) // 16

            def group_body(g, cur):
                be = g * 16
                dst16 = dst_v[pl.ds(be, 16)]
                for j in range(16):
                    e = be + j
                    dst_e = dst16[j]
                    ok = (e < nvalid) & (dst_e >= nlo)

                    @pl.when(ok & (dst_e != cur))
                    def _():
                        finalize(cur)

                    @pl.when(ok)
                    def _():
                        arow = h_v[e, pl.ds(K, 16)]
                        brow = ad_v[dst_e - nlo, :]
                        ev = arow + brow
                        ev = jnp.maximum(ev, LRELU * ev)
                        exv = jnp.exp(ev - mm)
                        s_v[...] = s_v[...] + exv
                        for jj in range(NV):
                            exs = exv[(jj * 16) // CPH]
                            acc_v[pl.ds(jj * 16, 16)] = (
                                acc_v[pl.ds(jj * 16, 16)]
                                + exs * h_v[e, pl.ds(jj * 16, 16)])
                    cur = jnp.where(ok, dst_e, cur)
                return cur

            return lax.fori_loop(0, ngroups, group_body, cur)

        cur = lax.fori_loop(0, nchunks, chunk_body, nlo)
        finalize(cur)
        pltpu.sync_copy(stg_v, out.at[pl.ds(pl.multiple_of(nlo, 8), NPT)])

    return k


def _proj(a):
    hh, cc = a.shape
    rows = jnp.arange(hh * cc)
    return jnp.where(jnp.arange(16)[None, :] == (rows // cc)[:, None],
                     a.reshape(-1)[:, None], 0.0).astype(jnp.float32)


def kernel(x, edge_index, W1, a_src1, a_dst1, b1, W2, a_src2, a_dst2, b2,
           W3, a_src3, a_dst3, b3):
    loop = jnp.arange(N, dtype=edge_index.dtype)
    src = jnp.concatenate([edge_index[0], loop])
    dst = jnp.concatenate([edge_index[1], loop])
    order = jnp.argsort(dst)
    ssrc = src[order].astype(jnp.int32)
    sdst = dst[order].astype(jnp.int32)
    ep = ssrc.shape[0]
    epad = ((ep + CE - 1) // CE + 1) * CE
    srcp = jnp.zeros((epad,), jnp.int32).at[:ep].set(ssrc)
    dstp = jnp.zeros((epad,), jnp.int32).at[:ep].set(sdst)
    marks = jnp.minimum(jnp.arange(NW + 1) * NPT, N).astype(jnp.int32)
    pos = jnp.searchsorted(sdst, marks).astype(jnp.int32)
    tecb = jnp.zeros((NW, 16), jnp.int32)
    tecb = (tecb.at[:, 0].set((pos[:NW] // 8) * 8)
                .at[:, 1].set(pos[1:])
                .at[:, 2].set(jnp.arange(NW, dtype=jnp.int32) * NPT)
            ).reshape(-1)

    ek128 = _edge_kernel(128, 256, HID, True)
    ek64 = _edge_kernel(64, 128, D_OUT, False)

    def gat(xin, W, a_s, a_d, b, ek, kg):
        hcat, a16d, m = _dense(xin, W, _proj(a_s), _proj(a_d), kg)
        adst_pad = jnp.zeros((NPAD, 16), jnp.float32).at[:N].set(a16d)
        out = ek(srcp, dstp, tecb, adst_pad, m.reshape(16), b, hcat)
        return out[:N]

    h1 = gat(x, W1, a_src1, a_dst1, b1, ek128, 256)
    h2 = gat(h1, W2, a_src2, a_dst2, b2, ek128, 256)
    h3 = gat(h2, W3, a_src3, a_dst3, b3, ek64, 128)
    return _log_softmax(h3)


# speedup vs baseline: 41.1291x; 2.6420x over previous
import functools

import jax
import jax.numpy as jnp
from jax import lax
from jax.experimental import pallas as pl
from jax.experimental.pallas import tpu as pltpu
from jax.experimental.pallas import tpu_sc as plsc

N = 10000
HEADS = 8
HID = 16
D_OUT = 64
NW = 32
NPT = 320
NPAD = NW * NPT
CE = 128
LRELU = 0.2
EPS = 1e-16


def _dense_body(kg, x_ref, w_ref, ps_ref, pd_ref, hc_ref, ad_ref, m_ref):
    n = x_ref.shape[0]
    h = jnp.dot(x_ref[...], w_ref[...], preferred_element_type=jnp.float32)
    a_s = jnp.dot(h, ps_ref[...], preferred_element_type=jnp.float32)
    a_d = jnp.dot(h, pd_ref[...], preferred_element_type=jnp.float32)
    k = h.shape[1]
    pad = kg - k - 16
    hc_ref[...] = jnp.concatenate(
        [h, a_s, jnp.zeros((n, pad), jnp.float32)], axis=1)
    ad_ref[...] = a_d
    t = jnp.max(a_s, axis=0, keepdims=True) + jnp.max(a_d, axis=0, keepdims=True)
    m_ref[...] = jnp.maximum(t, LRELU * t)


def _dense(x, w, ps, pd, kg):
    n = x.shape[0]
    return pl.pallas_call(
        functools.partial(_dense_body, kg),
        out_shape=(
            jax.ShapeDtypeStruct((n, kg), jnp.float32),
            jax.ShapeDtypeStruct((n, 16), jnp.float32),
            jax.ShapeDtypeStruct((1, 16), jnp.float32),
        ),
    )(x, w, ps, pd)


def _log_softmax_body(x_ref, o_ref):
    x = x_ref[...]
    m = jnp.max(x, axis=-1, keepdims=True)
    ex = jnp.exp(x - m)
    o_ref[...] = (x - m) - jnp.log(jnp.sum(ex, axis=-1, keepdims=True))


def _log_softmax(x):
    return pl.pallas_call(
        _log_softmax_body,
        out_shape=jax.ShapeDtypeStruct(x.shape, x.dtype),
    )(x)


def _edge_kernel(K, KG, CPH, relu):
    NV = K // 16
    mesh = plsc.VectorSubcoreMesh(core_axis_name="c", subcore_axis_name="s",
                                  num_cores=2, num_subcores=16)

    @functools.partial(
        pl.kernel,
        out_type=jax.ShapeDtypeStruct((NPAD, K), jnp.float32),
        mesh=mesh,
        scratch_types=[
            pltpu.VMEM((16,), jnp.int32),
            pltpu.VMEM((16,), jnp.float32),
            pltpu.VMEM((K,), jnp.float32),
            pltpu.VMEM((CE,), jnp.int32),
            pltpu.VMEM((CE,), jnp.int32),
            pltpu.VMEM((CE, KG), jnp.float32),
            pltpu.VMEM((NPT, 16), jnp.float32),
            pltpu.VMEM((NPT, K), jnp.float32),
            pltpu.SemaphoreType.DMA,
        ],
    )
    def k(srcp, dstp, tecb, adst, m16, bias, hcat, out,
          bounds_v, m_v, bias_v, src_v, dst_v, h_v, ad_v,
          stg_v, sem_h):
        t = lax.axis_index("s") * 2 + lax.axis_index("c")
        pltpu.sync_copy(tecb.at[pl.ds(pl.multiple_of(t * 16, 8), 16)], bounds_v)
        pltpu.sync_copy(m16, m_v)
        pltpu.sync_copy(bias, bias_v)
        bv = bounds_v[...]
        elo = bv[0]
        ehi = bv[1]
        nlo = bv[2]
        pltpu.sync_copy(adst.at[pl.ds(pl.multiple_of(nlo, 8), NPT)], ad_v)
        mm = m_v[...]
        zero16 = jnp.zeros((16,), jnp.float32)
        bias_r = [bias_v[pl.ds(j * 16, 16)] for j in range(NV)]

        def finalize(cur, s, accs):
            recip = 1.0 / (s + EPS)
            row = cur - nlo
            for j in range(NV):
                rj = recip[(j * 16) // CPH]
                ov = accs[j] * rj + bias_r[j]
                if relu:
                    ov = jnp.maximum(ov, 0.0)
                stg_v[row, pl.ds(j * 16, 16)] = ov

        nchunks = (ehi - elo + CE - 1) // CE

        def chunk_body(c, carry):
            base = pl.multiple_of(elo + c * CE, 8)
            pltpu.sync_copy(dstp.at[pl.ds(base, CE)], dst_v)
            pltpu.sync_copy(srcp.at[pl.ds(base, CE)], src_v)
            pltpu.async_copy(hcat.at[src_v], h_v, sem_h).wait()

            nvalid = jnp.minimum(CE, ehi - base)
            ngroups = (nvalid + 15) // 16

            def group_body(g, carry):
                cur = carry[0]
                s = carry[1]
                accs = list(carry[2:])
                be = g * 16
                dst16 = dst_v[pl.ds(be, 16)]
                for j in range(16):
                    e = be + j
                    dst_e = dst16[j]
                    ok = (e < nvalid) & (dst_e >= nlo)
                    bnd = ok & (dst_e != cur)

                    @pl.when(bnd)
                    def _(cur=cur, s=s, accs=tuple(accs)):
                        finalize(cur, s, accs)

                    row_i = jnp.minimum(jnp.maximum(dst_e - nlo, 0), NPT - 1)
                    arow = h_v[e, pl.ds(K, 16)]
                    brow = ad_v[row_i, :]
                    ev = arow + brow
                    ev = jnp.maximum(ev, LRELU * ev)
                    exv = jnp.exp(ev - mm)
                    exv = jnp.where(ok, exv, zero16)
                    s = jnp.where(bnd, exv, s + exv)
                    for jj in range(NV):
                        contrib = exv[(jj * 16) // CPH] * h_v[e, pl.ds(jj * 16, 16)]
                        accs[jj] = jnp.where(bnd, contrib, accs[jj] + contrib)
                    cur = jnp.where(ok, dst_e, cur)
                return (cur, s, *accs)

            return lax.fori_loop(0, ngroups, group_body, carry)

        carry0 = (nlo, zero16) + tuple(zero16 for _ in range(NV))
        carry = lax.fori_loop(0, nchunks, chunk_body, carry0)
        finalize(carry[0], carry[1], tuple(carry[2:]))
        pltpu.sync_copy(stg_v, out.at[pl.ds(pl.multiple_of(nlo, 8), NPT)])

    return k


def _proj(a):
    hh, cc = a.shape
    rows = jnp.arange(hh * cc)
    return jnp.where(jnp.arange(16)[None, :] == (rows // cc)[:, None],
                     a.reshape(-1)[:, None], 0.0).astype(jnp.float32)


def kernel(x, edge_index, W1, a_src1, a_dst1, b1, W2, a_src2, a_dst2, b2,
           W3, a_src3, a_dst3, b3):
    loop = jnp.arange(N, dtype=edge_index.dtype)
    src = jnp.concatenate([edge_index[0], loop])
    dst = jnp.concatenate([edge_index[1], loop])
    order = jnp.argsort(dst)
    ssrc = src[order].astype(jnp.int32)
    sdst = dst[order].astype(jnp.int32)
    ep = ssrc.shape[0]
    epad = ((ep + CE - 1) // CE + 1) * CE
    srcp = jnp.zeros((epad,), jnp.int32).at[:ep].set(ssrc)
    dstp = jnp.zeros((epad,), jnp.int32).at[:ep].set(sdst)
    marks = jnp.minimum(jnp.arange(NW + 1) * NPT, N).astype(jnp.int32)
    pos = jnp.searchsorted(sdst, marks).astype(jnp.int32)
    tecb = jnp.zeros((NW, 16), jnp.int32)
    tecb = (tecb.at[:, 0].set((pos[:NW] // 8) * 8)
                .at[:, 1].set(pos[1:])
                .at[:, 2].set(jnp.arange(NW, dtype=jnp.int32) * NPT)
            ).reshape(-1)

    ek128 = _edge_kernel(128, 256, HID, True)
    ek64 = _edge_kernel(64, 128, D_OUT, False)

    def gat(xin, W, a_s, a_d, b, ek, kg):
        hcat, a16d, m = _dense(xin, W, _proj(a_s), _proj(a_d), kg)
        adst_pad = jnp.zeros((NPAD, 16), jnp.float32).at[:N].set(a16d)
        out = ek(srcp, dstp, tecb, adst_pad, m.reshape(16), b, hcat)
        return out[:N]

    h1 = gat(x, W1, a_src1, a_dst1, b1, ek128, 256)
    h2 = gat(h1, W2, a_src2, a_dst2, b2, ek128, 256)
    h3 = gat(h2, W3, a_src3, a_dst3, b3, ek64, 128)
    return _log_softmax(h3)


# speedup vs baseline: 52.2340x; 1.2700x over previous
import functools

import jax
import jax.numpy as jnp
from jax import lax
from jax.experimental import pallas as pl
from jax.experimental.pallas import tpu as pltpu
from jax.experimental.pallas import tpu_sc as plsc

N = 10000
HEADS = 8
HID = 16
D_OUT = 64
NW = 32
NPT = 320
NPAD = NW * NPT
CE = 128
LRELU = 0.2
EPS = 1e-16


def _dense_body(kg, x_ref, w_ref, ps_ref, pd_ref, hc_ref, ad_ref, m_ref):
    n = x_ref.shape[0]
    h = jnp.dot(x_ref[...], w_ref[...], preferred_element_type=jnp.float32)
    a_s = jnp.dot(h, ps_ref[...], preferred_element_type=jnp.float32)
    a_d = jnp.dot(h, pd_ref[...], preferred_element_type=jnp.float32)
    k = h.shape[1]
    pad = kg - k - 16
    hc_ref[...] = jnp.concatenate(
        [h, a_s, jnp.zeros((n, pad), jnp.float32)], axis=1)
    ad_ref[...] = a_d
    t = jnp.max(a_s, axis=0, keepdims=True) + jnp.max(a_d, axis=0, keepdims=True)
    m_ref[...] = jnp.maximum(t, LRELU * t)


def _dense(x, w, ps, pd, kg):
    n = x.shape[0]
    return pl.pallas_call(
        functools.partial(_dense_body, kg),
        out_shape=(
            jax.ShapeDtypeStruct((n, kg), jnp.float32),
            jax.ShapeDtypeStruct((n, 16), jnp.float32),
            jax.ShapeDtypeStruct((1, 16), jnp.float32),
        ),
    )(x, w, ps, pd)


def _log_softmax_body(x_ref, o_ref):
    x = x_ref[...]
    m = jnp.max(x, axis=-1, keepdims=True)
    ex = jnp.exp(x - m)
    o_ref[...] = (x - m) - jnp.log(jnp.sum(ex, axis=-1, keepdims=True))


def _log_softmax(x):
    return pl.pallas_call(
        _log_softmax_body,
        out_shape=jax.ShapeDtypeStruct(x.shape, x.dtype),
    )(x)


def _edge_kernel(K, KG, CPH, relu):
    NV = K // 16
    mesh = plsc.VectorSubcoreMesh(core_axis_name="c", subcore_axis_name="s",
                                  num_cores=2, num_subcores=16)

    @functools.partial(
        pl.kernel,
        out_type=jax.ShapeDtypeStruct((NPAD, K), jnp.float32),
        mesh=mesh,
        scratch_types=[
            pltpu.VMEM((16,), jnp.int32),
            pltpu.VMEM((16,), jnp.float32),
            pltpu.VMEM((K,), jnp.float32),
            pltpu.VMEM((2 * CE,), jnp.int32),
            pltpu.VMEM((CE,), jnp.int32),
            pltpu.VMEM((CE, KG), jnp.float32),
            pltpu.VMEM((CE, KG), jnp.float32),
            pltpu.VMEM((NPT, 16), jnp.float32),
            pltpu.VMEM((160, K), jnp.float32),
            pltpu.SemaphoreType.DMA,
            pltpu.SemaphoreType.DMA,
        ],
    )
    def k(srcp, dstp, tecb, adst, m16, bias, hcat, out,
          bounds_v, m_v, bias_v, src_v, dst_v, h_a, h_b, ad_v,
          stg_v, sem_a, sem_b):
        t = lax.axis_index("s") * 2 + lax.axis_index("c")
        pltpu.sync_copy(tecb.at[pl.ds(pl.multiple_of(t * 16, 8), 16)], bounds_v)
        pltpu.sync_copy(m16, m_v)
        pltpu.sync_copy(bias, bias_v)
        bv = bounds_v[...]
        elo = bv[0]
        ehi = bv[1]
        nlo = bv[2]
        pltpu.sync_copy(adst.at[pl.ds(pl.multiple_of(nlo, 8), NPT)], ad_v)
        mm = m_v[...]
        zero16 = jnp.zeros((16,), jnp.float32)
        bias_r = [bias_v[pl.ds(j * 16, 16)] for j in range(NV)]

        def finalize(cur, s, accs):
            recip = 1.0 / (s + EPS)
            row = cur - nlo
            rr = row % 160
            for j in range(NV):
                rj = recip[(j * 16) // CPH]
                ov = accs[j] * rj + bias_r[j]
                if relu:
                    ov = jnp.maximum(ov, 0.0)
                stg_v[rr, pl.ds(j * 16, 16)] = ov

            @pl.when(row % 80 == 79)
            def _():
                bb = rr - 79
                fs = nlo + row - 79
                pltpu.sync_copy(
                    stg_v.at[pl.ds(pl.multiple_of(bb, 8), 80)],
                    out.at[pl.ds(pl.multiple_of(fs, 8), 80)])

        nchunks = (ehi - elo + CE - 1) // CE
        npairs = (nchunks + 1) // 2

        def load_idx(p):
            @pl.when(2 * p < nchunks)
            def _():
                base = pl.multiple_of(elo + p * 2 * CE, 8)
                pltpu.sync_copy(srcp.at[pl.ds(base, 2 * CE)], src_v)

        def start_g(c, off, hb, sem):
            @pl.when(c < nchunks)
            def _():
                pltpu.async_copy(hcat.at[src_v.at[pl.ds(off, CE)]], hb, sem)

        def wait_g(c, off, hb, sem):
            @pl.when(c < nchunks)
            def _():
                pltpu.make_async_copy(
                    hcat.at[src_v.at[pl.ds(off, CE)]], hb, sem).wait()

        def process(c, off, hb, carry):
            base = elo + c * CE
            nvalid = jnp.minimum(CE, ehi - base)
            ngroups = (nvalid + 15) // 16

            @pl.when(c < nchunks)
            def _():
                pltpu.sync_copy(dstp.at[pl.ds(pl.multiple_of(base, 8), CE)],
                                dst_v)

            def group_body(g, carry):
                cur = carry[0]
                s = carry[1]
                accs = list(carry[2:])
                be = g * 16
                dst16 = dst_v[pl.ds(be, 16)]
                for j in range(16):
                    e = be + j
                    dst_e = dst16[j]
                    ok = (e < nvalid) & (dst_e >= nlo)
                    bnd = ok & (dst_e != cur)

                    @pl.when(bnd)
                    def _(cur=cur, s=s, accs=tuple(accs)):
                        finalize(cur, s, accs)

                    row_i = jnp.minimum(jnp.maximum(dst_e - nlo, 0), NPT - 1)
                    arow = hb[e, pl.ds(K, 16)]
                    brow = ad_v[row_i, :]
                    ev = arow + brow
                    ev = jnp.maximum(ev, LRELU * ev)
                    exv = jnp.exp(ev - mm)
                    exv = jnp.where(ok, exv, zero16)
                    s = jnp.where(bnd, exv, s + exv)
                    for jj in range(NV):
                        contrib = exv[(jj * 16) // CPH] * hb[e, pl.ds(jj * 16, 16)]
                        accs[jj] = jnp.where(bnd, contrib, accs[jj] + contrib)
                    cur = jnp.where(ok, dst_e, cur)
                return (cur, s, *accs)

            return lax.fori_loop(0, ngroups, group_body, carry)

        load_idx(0)
        start_g(0, 0, h_a, sem_a)

        def pair_body(i, carry):
            c0 = 2 * i
            start_g(c0 + 1, CE, h_b, sem_b)
            wait_g(c0, 0, h_a, sem_a)
            carry = process(c0, 0, h_a, carry)
            wait_g(c0 + 1, CE, h_b, sem_b)
            load_idx(i + 1)
            start_g(c0 + 2, 0, h_a, sem_a)
            carry = process(c0 + 1, CE, h_b, carry)
            return carry

        carry0 = (nlo, zero16) + tuple(zero16 for _ in range(NV))
        carry = lax.fori_loop(0, npairs, pair_body, carry0)
        finalize(carry[0], carry[1], tuple(carry[2:]))

    return k


def _proj(a):
    hh, cc = a.shape
    rows = jnp.arange(hh * cc)
    return jnp.where(jnp.arange(16)[None, :] == (rows // cc)[:, None],
                     a.reshape(-1)[:, None], 0.0).astype(jnp.float32)


def kernel(x, edge_index, W1, a_src1, a_dst1, b1, W2, a_src2, a_dst2, b2,
           W3, a_src3, a_dst3, b3):
    loop = jnp.arange(N, dtype=edge_index.dtype)
    src = jnp.concatenate([edge_index[0], loop])
    dst = jnp.concatenate([edge_index[1], loop])
    order = jnp.argsort(dst)
    ssrc = src[order].astype(jnp.int32)
    sdst = dst[order].astype(jnp.int32)
    ep = ssrc.shape[0]
    epad = ((ep + CE - 1) // CE + 1) * CE
    srcp = jnp.zeros((epad,), jnp.int32).at[:ep].set(ssrc)
    dstp = jnp.zeros((epad,), jnp.int32).at[:ep].set(sdst)
    marks = jnp.minimum(jnp.arange(NW + 1) * NPT, N).astype(jnp.int32)
    pos = jnp.searchsorted(sdst, marks).astype(jnp.int32)
    tecb = jnp.zeros((NW, 16), jnp.int32)
    tecb = (tecb.at[:, 0].set((pos[:NW] // 8) * 8)
                .at[:, 1].set(pos[1:])
                .at[:, 2].set(jnp.arange(NW, dtype=jnp.int32) * NPT)
            ).reshape(-1)

    ek128 = _edge_kernel(128, 256, HID, True)
    ek64 = _edge_kernel(64, 128, D_OUT, False)

    def gat(xin, W, a_s, a_d, b, ek, kg):
        hcat, a16d, m = _dense(xin, W, _proj(a_s), _proj(a_d), kg)
        adst_pad = jnp.zeros((NPAD, 16), jnp.float32).at[:N].set(a16d)
        out = ek(srcp, dstp, tecb, adst_pad, m.reshape(16), b, hcat)
        return out[:N]

    h1 = gat(x, W1, a_src1, a_dst1, b1, ek128, 256)
    h2 = gat(h1, W2, a_src2, a_dst2, b2, ek128, 256)
    h3 = gat(h2, W3, a_src3, a_dst3, b3, ek64, 128)
    return _log_softmax(h3)
